# Initial kernel scaffold; baseline (speedup 1.0000x reference)
import jax
import jax.numpy as jnp
from jax.experimental import pallas as pl


def kernel(x, edge_index, edge_attr, batch, y, params):
    raise NotImplementedError("write your pallas kernel here")


# speedup vs baseline: 9.7332x; 9.7332x over previous
import functools

import jax
import jax.numpy as jnp
from jax import lax
from jax.experimental import pallas as pl
from jax.experimental.pallas import tpu as pltpu
from jax.experimental.pallas import tpu_sc as plsc

N = 50000
E = 800000
G = 512
C = 64

NC = 2
NS = 16
NW = NC * NS

EPAD = 819200
ER = EPAD // 128
PADE = EPAD - E
BIG = 2 ** 30

HALF = N // 2
HPAD = 25088
STRIPE = HPAD // NS

_mesh = plsc.VectorSubcoreMesh(core_axis_name="c", subcore_axis_name="s",
                               num_cores=NC, num_subcores=NS)
_SC_PARAMS = pltpu.CompilerParams(use_tc_tiling_on_sc=False,
                                  needs_layout_passes=False)

f32 = jnp.float32
i32 = jnp.int32


def _zero_rows(buf, nrows, d):
  z = jnp.zeros((16,), f32)

  def zr(i, _):
    for cc in range(d // 16):
      buf[i, pl.ds(cc * 16, 16)] = z
    return 0

  lax.fori_loop(0, nrows, zr, 0)


def _edge_pass_body(d, crows, linear_src, scale, src, *rest):
  if linear_src:
    (sidx, out, gbuf, sidxb, locb, acc) = rest
    gidx = exh = exb = None
  else:
    if scale:
      (gidx, sidx, exh, out, gbuf, gidxb, sidxb, locb, exb, acc) = rest
    else:
      (gidx, sidx, out, gbuf, gidxb, sidxb, locb, acc) = rest
      exh = exb = None

  c = lax.axis_index("c")
  s = lax.axis_index("s")
  lo = c * HALF
  it = lax.iota(i32, 16)
  zr = crows * 128

  _zero_rows(gbuf, zr, d)
  base = s * STRIPE
  for q in range(STRIPE // zr):
    pltpu.sync_copy(gbuf, acc.at[pl.ds(base + q * zr, zr)])
  rem = STRIPE % zr
  if rem:
    pltpu.sync_copy(gbuf.at[pl.ds(0, rem)],
                    acc.at[pl.ds(base + (STRIPE // zr) * zr, rem)])
  plsc.subcore_barrier()

  def chunk(k, _):
    rowb = s * 400 + k * crows
    pltpu.sync_copy(sidx.at[pl.ds(rowb, crows)], sidxb)
    if linear_src:
      pltpu.sync_copy(src.at[pl.ds(rowb * 128, zr)], gbuf)
    else:
      pltpu.sync_copy(gidx.at[pl.ds(rowb, crows)], gidxb)
      for j in range(crows):
        pltpu.sync_copy(src.at[gidxb.at[j]], gbuf.at[pl.ds(j * 128, 128)])
    if scale:
      pltpu.sync_copy(exh.at[pl.ds(rowb, crows)], exb)

      def sg(gi, _):
        j = gi // 8
        gg = (gi % 8) * 16
        ev = exb[j, pl.ds(gg, 16)]
        for l in range(16):
          val = ev[l]
          r = j * 128 + gg + l
          for cc in range(d // 16):
            sl = pl.ds(cc * 16, 16)
            gbuf[r, sl] = gbuf[r, sl] * jnp.full((16,), val)
        return 0

      lax.fori_loop(0, crows * 8, sg, 0)

    def rg(g, _):
      j = g // 8
      t = (g % 8) * 16
      r = sidxb[j, pl.ds(t, 16)]
      loc = r - lo
      ok = (loc >= 0) & (loc < HALF)
      garb = HALF + ((g * 16 + it) & 63)
      locb[j, pl.ds(t, 16)] = jnp.where(ok, loc, garb)
      return 0

    lax.fori_loop(0, crows * 8, rg, 0)
    for j in range(crows):
      pltpu.sync_copy(gbuf.at[pl.ds(j * 128, 128)], acc.at[locb.at[j]],
                      add=True)
    return 0

  lax.fori_loop(0, 400 // crows, chunk, 0)
  plsc.subcore_barrier()

  nfull = HALF // zr
  wrem = HALF % zr

  def wb(i, _):
    @pl.when((i % 16) == s)
    def _():
      pltpu.sync_copy(acc.at[pl.ds(i * zr, zr)], gbuf)
      pltpu.sync_copy(gbuf, out.at[pl.ds(lo + i * zr, zr)])
    return 0

  lax.fori_loop(0, nfull, wb, 0)
  if wrem:
    @pl.when(s == 15)
    def _():
      pltpu.sync_copy(acc.at[pl.ds(nfull * zr, wrem)],
                      gbuf.at[pl.ds(0, wrem)])
      pltpu.sync_copy(gbuf.at[pl.ds(0, wrem)],
                      out.at[pl.ds(lo + nfull * zr, wrem)])


def _make_edge_pass(d, crows, linear_src, scale):
  scratch = [pltpu.VMEM((crows * 128, d), f32)]
  if not linear_src:
    scratch.append(pltpu.VMEM((crows, 128), i32))
  scratch.append(pltpu.VMEM((crows, 128), i32))
  scratch.append(pltpu.VMEM((crows, 128), i32))
  if scale:
    scratch.append(pltpu.VMEM((crows, 128), f32))
  scratch.append(pltpu.VMEM_SHARED((HPAD, d), f32))
  return pl.kernel(
      functools.partial(_edge_pass_body, d, crows, linear_src, scale),
      out_type=jax.ShapeDtypeStruct((N, d), f32),
      mesh=_mesh, scratch_types=scratch,
      compiler_params=_SC_PARAMS)


_ep_lin16 = _make_edge_pass(16, 8, True, False)
_ep_gs16 = _make_edge_pass(16, 8, False, False)
_ep_gs64 = _make_edge_pass(64, 2, False, False)
_ep_gs48s = _make_edge_pass(48, 4, False, True)
_ep_gs32s = _make_edge_pass(32, 8, False, True)


def _gatp1_body(aa, gidx, didx, mg, exo, aav, rb, cb, eb, mgv):
  c = lax.axis_index("c")
  s = lax.axis_index("s")
  wid = s * NC + c
  pltpu.sync_copy(aa, aav)
  pltpu.sync_copy(mg, mgv)
  mgvec = mgv[...]
  rowbase = wid * 200

  def chunk(k, _):
    rowb = rowbase + k * 8
    pltpu.sync_copy(gidx.at[pl.ds(rowb, 8)], rb)
    pltpu.sync_copy(didx.at[pl.ds(rowb, 8)], cb)

    def g(gi, _):
      j = gi // 8
      t = (gi % 8) * 16
      r = rb[j, pl.ds(t, 16)]
      cc = cb[j, pl.ds(t, 16)]
      a_s = plsc.load_gather(aav, [r * 2])
      a_d = plsc.load_gather(aav, [cc * 2 + 1])
      e = a_s + a_d
      e = jnp.where(e >= 0.0, e, e * 0.2)
      eb[j, pl.ds(t, 16)] = jnp.exp(e - mgvec)
      return 0

    lax.fori_loop(0, 64, g, 0)
    pltpu.sync_copy(eb, exo.at[pl.ds(rowb, 8)])
    return 0

  lax.fori_loop(0, 25, chunk, 0)


_gatp1 = pl.kernel(
    _gatp1_body,
    out_type=jax.ShapeDtypeStruct((ER, 128), f32),
    mesh=_mesh,
    scratch_types=[pltpu.VMEM((2 * N,), f32), pltpu.VMEM((8, 128), i32),
                   pltpu.VMEM((8, 128), i32), pltpu.VMEM((8, 128), f32),
                   pltpu.VMEM((16,), f32)],
    compiler_params=_SC_PARAMS)


def _bmean_body(xnew, batch, po, gacc, rbuf, bb):
  c = lax.axis_index("c")
  s = lax.axis_index("s")
  wid = s * NC + c
  z = jnp.zeros((16,), f32)

  def zr(i, _):
    gacc[pl.ds(i * 16, 16)] = z
    return 0

  lax.fori_loop(0, G * 80 // 16, zr, 0)
  onev = jnp.where(lax.iota(i32, 16) == 0, 1.0, 0.0).astype(f32)

  def chunkq(q, _):
    i = wid + q * 32

    @pl.when(i < 125)
    def _():
      pltpu.sync_copy(batch.at[pl.ds(i * 400, 400)], bb)
      pltpu.sync_copy(xnew.at[pl.ds(i * 400, 400)], rbuf)

      def ngrp(g, _):
        bv = bb[pl.ds(g * 16, 16)]
        for l in range(16):
          b = bv[l]
          base = b * 80
          e = g * 16 + l
          for cc in range(4):
            sl = pl.ds(base + cc * 16, 16)
            gacc[sl] = gacc[sl] + rbuf[e, pl.ds(cc * 16, 16)]
          slc = pl.ds(base + 64, 16)
          gacc[slc] = gacc[slc] + onev
        return 0

      lax.fori_loop(0, 25, ngrp, 0)
    return 0

  lax.fori_loop(0, 4, chunkq, 0)
  pltpu.sync_copy(gacc, po.at[pl.ds(wid * (G * 80), G * 80)])


_bmean = pl.kernel(
    _bmean_body,
    out_type=jax.ShapeDtypeStruct((NW * G * 80,), f32),
    mesh=_mesh,
    scratch_types=[pltpu.VMEM((G * 80,), f32), pltpu.VMEM((400, 64), f32),
                   pltpu.VMEM((400,), i32)],
    compiler_params=_SC_PARAMS)


def _expand_body(ug, batch, ubx, ugv, obuf, bb):
  c = lax.axis_index("c")
  s = lax.axis_index("s")
  wid = s * NC + c
  pltpu.sync_copy(ug, ugv)

  def chunkq(q, _):
    i = wid + q * 32

    @pl.when(i < 125)
    def _():
      pltpu.sync_copy(batch.at[pl.ds(i * 400, 400)], bb)

      def ngrp(g, _):
        bv = bb[pl.ds(g * 16, 16)]
        for l in range(16):
          b = bv[l]
          base = b * 64
          e = g * 16 + l
          for cc in range(4):
            obuf[e, pl.ds(cc * 16, 16)] = ugv[pl.ds(base + cc * 16, 16)]
        return 0

      lax.fori_loop(0, 25, ngrp, 0)
      pltpu.sync_copy(obuf, ubx.at[pl.ds(i * 400, 400)])
    return 0

  lax.fori_loop(0, 4, chunkq, 0)


_expand = pl.kernel(
    _expand_body,
    out_type=jax.ShapeDtypeStruct((N, 64), f32),
    mesh=_mesh,
    scratch_types=[pltpu.VMEM((G * 64,), f32), pltpu.VMEM((400, 64), f32),
                   pltpu.VMEM((400,), i32)],
    compiler_params=_SC_PARAMS)


BN = 2000
GRID = N // BN


def _dot(a, b):
  return jnp.dot(a, b, preferred_element_type=f32)


def _c1_body(s16, t16, w1x, wt, w2a, b2, o):
  t = t16[...]
  o1 = (_dot(s16[...], w1x[...]) + _dot(t, wt[...]))
  o1 = o1 / jnp.maximum(t[:, 4:5], 1.0)
  o[...] = _dot(o1, w2a[...]) + b2[...]


_tc_c1 = pl.pallas_call(
    _c1_body,
    grid=(GRID,),
    in_specs=[pl.BlockSpec((BN, 16), lambda i: (i, 0)),
              pl.BlockSpec((BN, 16), lambda i: (i, 0)),
              pl.BlockSpec((16, 64), lambda i: (0, 0)),
              pl.BlockSpec((16, 64), lambda i: (0, 0)),
              pl.BlockSpec((64, 64), lambda i: (0, 0)),
              pl.BlockSpec((1, 64), lambda i: (0, 0))],
    out_specs=pl.BlockSpec((BN, 64), lambda i: (i, 0)),
    out_shape=jax.ShapeDtypeStruct((N, 64), f32))


def _gatdense_body(outp, x16, w64, wx16, a2, hh1, hh2, aa, pmax):
  h = _dot(outp[...], w64[...]) + _dot(x16[...], wx16[...])
  av = _dot(h, a2[...])
  hh1[...] = h[:, :48]
  hh2[...] = jnp.concatenate(
      [h[:, 48:], jnp.ones((BN, 1), f32), jnp.zeros((BN, 15), f32)], axis=1)
  aa[...] = av
  m0 = jnp.max(av[:, 0])
  m1 = jnp.max(av[:, 1])
  pmax[...] = jnp.concatenate(
      [jnp.full((1, 1, 1), m0, f32), jnp.full((1, 1, 1), m1, f32),
       jnp.zeros((1, 1, 126), f32)], axis=2)


_tc_gatdense = pl.pallas_call(
    _gatdense_body,
    grid=(GRID,),
    in_specs=[pl.BlockSpec((BN, 64), lambda i: (i, 0)),
              pl.BlockSpec((BN, 16), lambda i: (i, 0)),
              pl.BlockSpec((64, 64), lambda i: (0, 0)),
              pl.BlockSpec((16, 64), lambda i: (0, 0)),
              pl.BlockSpec((64, 2), lambda i: (0, 0))],
    out_specs=[pl.BlockSpec((BN, 48), lambda i: (i, 0)),
               pl.BlockSpec((BN, 32), lambda i: (i, 0)),
               pl.BlockSpec((BN, 2), lambda i: (i, 0)),
               pl.BlockSpec((1, 1, 128), lambda i: (i, 0, 0))],
    out_shape=[jax.ShapeDtypeStruct((N, 48), f32),
               jax.ShapeDtypeStruct((N, 32), f32),
               jax.ShapeDtypeStruct((N, 2), f32),
               jax.ShapeDtypeStruct((GRID, 1, 128), f32)])


def _gout_body(s48, s32, bg, o):
  a = jnp.concatenate([s48[...], s32[..., :16]], axis=1)
  den = s32[..., 16:17]
  o[...] = a / jnp.maximum(den, 1e-16) + bg[...]


_tc_gout = pl.pallas_call(
    _gout_body,
    grid=(GRID,),
    in_specs=[pl.BlockSpec((BN, 48), lambda i: (i, 0)),
              pl.BlockSpec((BN, 32), lambda i: (i, 0)),
              pl.BlockSpec((1, 64), lambda i: (0, 0))],
    out_specs=pl.BlockSpec((BN, 64), lambda i: (i, 0)),
    out_shape=jax.ShapeDtypeStruct((N, 64), f32))


def _myc_body(sm, t16, ubx, outp, wa, wt, w2a, b2, xn, on):
  t = t16[...]
  o1 = _dot(sm[...], wa[...]) + _dot(t, wt[...])
  o1 = o1 / jnp.maximum(t[:, 4:5], 1.0)
  x = _dot(o1, w2a[...]) + ubx[...] + b2[...]
  xn[...] = x
  on[...] = outp[...] + x


_tc_myc = pl.pallas_call(
    _myc_body,
    grid=(GRID,),
    in_specs=[pl.BlockSpec((BN, 64), lambda i: (i, 0)),
              pl.BlockSpec((BN, 16), lambda i: (i, 0)),
              pl.BlockSpec((BN, 64), lambda i: (i, 0)),
              pl.BlockSpec((BN, 64), lambda i: (i, 0)),
              pl.BlockSpec((64, 64), lambda i: (0, 0)),
              pl.BlockSpec((16, 64), lambda i: (0, 0)),
              pl.BlockSpec((64, 64), lambda i: (0, 0)),
              pl.BlockSpec((1, 64), lambda i: (0, 0))],
    out_specs=[pl.BlockSpec((BN, 64), lambda i: (i, 0)),
               pl.BlockSpec((BN, 64), lambda i: (i, 0))],
    out_shape=[jax.ShapeDtypeStruct((N, 64), f32),
               jax.ShapeDtypeStruct((N, 64), f32)])


def _uupd_body(p, u, wg, bg, w2bn, gn, ugn):
  ps = jnp.sum(p[...], axis=0)
  sb = ps[:, :64] / jnp.maximum(ps[:, 64:65], 1.0)
  uu = u[...]
  unew = _dot(jnp.concatenate([uu, sb], axis=1), wg[...]) + bg[...]
  g = uu + unew
  gn[...] = g
  ugn[...] = _dot(g, w2bn[...])


_tc_uupd = pl.pallas_call(
    _uupd_body,
    in_specs=[pl.BlockSpec((NW, G, 80), lambda: (0, 0, 0)),
              pl.BlockSpec((G, 64), lambda: (0, 0)),
              pl.BlockSpec((128, 64), lambda: (0, 0)),
              pl.BlockSpec((1, 64), lambda: (0, 0)),
              pl.BlockSpec((64, 64), lambda: (0, 0))],
    out_specs=[pl.BlockSpec((G, 64), lambda: (0, 0)),
               pl.BlockSpec((G, 64), lambda: (0, 0))],
    out_shape=[jax.ShapeDtypeStruct((G, 64), f32),
               jax.ShapeDtypeStruct((G, 64), f32)])


def _head_body(g, w1, b1, w2, b2, w3, b3, o):
  p = _dot(g[...], w1[...]) + b1[...]
  p = jnp.where(p >= 0.0, p, 0.01 * p)
  p = _dot(p, w2[...]) + b2[...]
  p = jnp.where(p >= 0.0, p, 0.01 * p)
  o[...] = _dot(p, w3[...]) + b3[...]


_tc_head = pl.pallas_call(
    _head_body,
    in_specs=[pl.BlockSpec((G, 64), lambda: (0, 0)),
              pl.BlockSpec((64, 64), lambda: (0, 0)),
              pl.BlockSpec((1, 64), lambda: (0, 0)),
              pl.BlockSpec((64, 32), lambda: (0, 0)),
              pl.BlockSpec((1, 32), lambda: (0, 0)),
              pl.BlockSpec((32, 1), lambda: (0, 0)),
              pl.BlockSpec((1, 1), lambda: (0, 0))],
    out_specs=pl.BlockSpec((G, 1), lambda: (0, 0)),
    out_shape=jax.ShapeDtypeStruct((G, 1), f32))


def kernel(x, edge_index, edge_attr, batch, y, params):
  row = edge_index[0].astype(i32)
  col = edge_index[1].astype(i32)
  pad_g = (jnp.arange(PADE, dtype=i32) * 61) % N
  pad_s = jnp.full((PADE,), BIG, i32)
  row_g = jnp.concatenate([row, pad_g]).reshape(ER, 128)
  col_g = jnp.concatenate([col, pad_g]).reshape(ER, 128)
  row_s = jnp.concatenate([row, pad_s]).reshape(ER, 128)
  col_s = jnp.concatenate([col, pad_s]).reshape(ER, 128)

  ea16 = jnp.zeros((EPAD, 16), f32)
  ea16 = ea16.at[:E, :4].set(edge_attr).at[:E, 4].set(1.0)
  x16 = jnp.zeros((N, 16), f32).at[:, :2].set(x)

  t16 = _ep_lin16(ea16, row_s)
  s16 = _ep_gs16(x16, col_g, row_s)

  p1 = params["conv1"]
  w1x16 = jnp.zeros((16, 64), f32).at[:2].set(p1["W1"][:2])
  wt1 = jnp.zeros((16, 64), f32).at[:4].set(p1["W1"][2:6]).at[4].set(p1["b1"])
  out = _tc_c1(s16, t16, w1x16, wt1, p1["W2"][:64], p1["b2"][None])

  pacc = _bmean(out, batch).reshape(NW, G, 80)
  glob, ug = _tc_uupd(pacc, jnp.zeros((G, 64), f32), p1["Wg"], p1["bg"][None],
                      params["convs"][0]["W2"][64:])

  for i in range(4):
    gp = params["gats"][i]
    mp = params["convs"][i]
    wg64 = gp["W"][:64]
    wgx16 = jnp.zeros((16, 64), f32).at[:2].set(gp["W"][64:66])
    a2 = jnp.stack([gp["asrc"], gp["adst"]], axis=1)
    hh1, hh2, aa, pmax = _tc_gatdense(out, x16, wg64, wgx16, a2)
    mg = jax.nn.leaky_relu(jnp.max(pmax[:, 0, 0]) + jnp.max(pmax[:, 0, 1]),
                           0.2)
    mg16 = jnp.full((16,), mg, f32)
    ex = _gatp1(aa.reshape(-1), row_g, col_g, mg16)
    s48 = _ep_gs48s(hh1, row_g, col_s, ex)
    s32 = _ep_gs32s(hh2, row_g, col_s, ex)
    gout = _tc_gout(s48, s32, gp["b"][None])
    sm = _ep_gs64(gout, col_g, row_s)
    ubx = _expand(ug.reshape(-1), batch)
    wtm = (jnp.zeros((16, 64), f32).at[:4].set(mp["W1"][64:68])
           .at[4].set(mp["b1"]))
    xnew, out = _tc_myc(sm, t16, ubx, out, mp["W1"][:64], wtm,
                        mp["W2"][:64], mp["b2"][None])
    pacc = _bmean(xnew, batch).reshape(NW, G, 80)
    w2bn = (params["convs"][i + 1]["W2"][64:] if i < 3
            else jnp.zeros((64, 64), f32))
    glob, ug = _tc_uupd(pacc, glob, mp["Wg"], mp["bg"][None], w2bn)

  po = params["out"]
  pred = _tc_head(glob, po["W1"], po["b1"][None], po["W2"], po["b2"][None],
                  po["W3"], po["b3"][None])
  return jnp.squeeze(pred, axis=-1)


# speedup vs baseline: 12.9859x; 1.3342x over previous
import functools

import jax
import jax.numpy as jnp
from jax import lax
from jax.experimental import pallas as pl
from jax.experimental.pallas import tpu as pltpu
from jax.experimental.pallas import tpu_sc as plsc

N = 50000
E = 800000
G = 512
C = 64

NC = 2
NS = 16
NW = NC * NS

EPAD = 819200
ER = EPAD // 128
PADE = EPAD - E
BIG = 2 ** 30

HALF = N // 2
HPAD = 25088
STRIPE = HPAD // NS

_mesh = plsc.VectorSubcoreMesh(core_axis_name="c", subcore_axis_name="s",
                               num_cores=NC, num_subcores=NS)
_SC_PARAMS = pltpu.CompilerParams(use_tc_tiling_on_sc=False,
                                  needs_layout_passes=False)

f32 = jnp.float32
i32 = jnp.int32


def _zero_rows(buf, nrows, d):
  z = jnp.zeros((16,), f32)

  def zr(i, _):
    for cc in range(d // 16):
      buf[i, pl.ds(cc * 16, 16)] = z
    return 0

  lax.fori_loop(0, nrows, zr, 0)


def _edge_pass_body(d, crows, linear_src, scale, src, *rest):
  if linear_src:
    (sidx, out, gbuf, sidxb, locb, semg, sems, acc) = rest
    gidx = exh = exb = None
  else:
    if scale:
      (gidx, sidx, exh, out, gbuf, gidxb, sidxb, locb, exb, semg, sems,
       acc) = rest
    else:
      (gidx, sidx, out, gbuf, gidxb, sidxb, locb, semg, sems, acc) = rest
      exh = exb = None

  c = lax.axis_index("c")
  s = lax.axis_index("s")
  lo = c * HALF
  it = lax.iota(i32, 16)
  zr = crows * 128

  _zero_rows(gbuf, zr, d)
  base = s * STRIPE
  for q in range(STRIPE // zr):
    pltpu.sync_copy(gbuf, acc.at[pl.ds(base + q * zr, zr)])
  rem = STRIPE % zr
  if rem:
    pltpu.sync_copy(gbuf.at[pl.ds(0, rem)],
                    acc.at[pl.ds(base + (STRIPE // zr) * zr, rem)])
  plsc.subcore_barrier()

  def chunk(k, _):
    rowb = s * 400 + k * crows
    pltpu.sync_copy(sidx.at[pl.ds(rowb, crows)], sidxb)
    if linear_src:
      pltpu.sync_copy(src.at[pl.ds(rowb * 128, zr)], gbuf)
      gd = []
    else:
      pltpu.sync_copy(gidx.at[pl.ds(rowb, crows)], gidxb)
      gd = [pltpu.async_copy(src.at[gidxb.at[j]],
                             gbuf.at[pl.ds(j * 128, 128)], semg)
            for j in range(crows)]
    if scale:
      pltpu.sync_copy(exh.at[pl.ds(rowb, crows)], exb)

      def sg(gi, _):
        j = gi // 8
        gg = (gi % 8) * 16
        ev = exb[j, pl.ds(gg, 16)]
        for l in range(16):
          val = ev[l]
          r = j * 128 + gg + l
          for cc in range(d // 16):
            sl = pl.ds(cc * 16, 16)
            gbuf[r, sl] = gbuf[r, sl] * jnp.full((16,), val)
        return 0

    def rg(g, _):
      j = g // 8
      t = (g % 8) * 16
      r = sidxb[j, pl.ds(t, 16)]
      loc = r - lo
      ok = (loc >= 0) & (loc < HALF)
      garb = HALF + ((g * 16 + it) & 63)
      locb[j, pl.ds(t, 16)] = jnp.where(ok, loc, garb)
      return 0

    lax.fori_loop(0, crows * 8, rg, 0)
    for dsc in gd:
      dsc.wait()
    if scale:
      lax.fori_loop(0, crows * 8, sg, 0)
    sd = [pltpu.async_copy(gbuf.at[pl.ds(j * 128, 128)], acc.at[locb.at[j]],
                           sems, add=True) for j in range(crows)]
    for dsc in sd:
      dsc.wait()
    return 0

  lax.fori_loop(0, 400 // crows, chunk, 0)
  plsc.subcore_barrier()

  nfull = HALF // zr
  wrem = HALF % zr

  def wb(i, _):
    @pl.when((i % 16) == s)
    def _():
      pltpu.sync_copy(acc.at[pl.ds(i * zr, zr)], gbuf)
      pltpu.sync_copy(gbuf, out.at[pl.ds(lo + i * zr, zr)])
    return 0

  lax.fori_loop(0, nfull, wb, 0)
  if wrem:
    @pl.when(s == 15)
    def _():
      pltpu.sync_copy(acc.at[pl.ds(nfull * zr, wrem)],
                      gbuf.at[pl.ds(0, wrem)])
      pltpu.sync_copy(gbuf.at[pl.ds(0, wrem)],
                      out.at[pl.ds(lo + nfull * zr, wrem)])


def _make_edge_pass(d, crows, linear_src, scale):
  scratch = [pltpu.VMEM((crows * 128, d), f32)]
  if not linear_src:
    scratch.append(pltpu.VMEM((crows, 128), i32))
  scratch.append(pltpu.VMEM((crows, 128), i32))
  scratch.append(pltpu.VMEM((crows, 128), i32))
  if scale:
    scratch.append(pltpu.VMEM((crows, 128), f32))
  scratch.append(pltpu.SemaphoreType.DMA)
  scratch.append(pltpu.SemaphoreType.DMA)
  scratch.append(pltpu.VMEM_SHARED((HPAD, d), f32))
  return pl.kernel(
      functools.partial(_edge_pass_body, d, crows, linear_src, scale),
      out_type=jax.ShapeDtypeStruct((N, d), f32),
      mesh=_mesh, scratch_types=scratch,
      compiler_params=_SC_PARAMS)


_ep_lin16 = _make_edge_pass(16, 8, True, False)
_ep_gs16 = _make_edge_pass(16, 8, False, False)
_ep_gs32 = _make_edge_pass(32, 8, False, False)
_ep_gs48s = _make_edge_pass(48, 4, False, True)
_ep_gs32s = _make_edge_pass(32, 8, False, True)


def _gatp1_body(aa, gidx, didx, mg, exo, aav, rb, cb, eb, mgv):
  c = lax.axis_index("c")
  s = lax.axis_index("s")
  wid = s * NC + c
  pltpu.sync_copy(aa, aav)
  pltpu.sync_copy(mg, mgv)
  mgvec = mgv[...]
  rowbase = wid * 200

  def chunk(k, _):
    rowb = rowbase + k * 8
    pltpu.sync_copy(gidx.at[pl.ds(rowb, 8)], rb)
    pltpu.sync_copy(didx.at[pl.ds(rowb, 8)], cb)

    def g(gi, _):
      j = gi // 8
      t = (gi % 8) * 16
      r = rb[j, pl.ds(t, 16)]
      cc = cb[j, pl.ds(t, 16)]
      a_s = plsc.load_gather(aav, [r * 2])
      a_d = plsc.load_gather(aav, [cc * 2 + 1])
      e = a_s + a_d
      e = jnp.where(e >= 0.0, e, e * 0.2)
      eb[j, pl.ds(t, 16)] = jnp.exp(e - mgvec)
      return 0

    lax.fori_loop(0, 64, g, 0)
    pltpu.sync_copy(eb, exo.at[pl.ds(rowb, 8)])
    return 0

  lax.fori_loop(0, 25, chunk, 0)


_gatp1 = pl.kernel(
    _gatp1_body,
    out_type=jax.ShapeDtypeStruct((ER, 128), f32),
    mesh=_mesh,
    scratch_types=[pltpu.VMEM((2 * N,), f32), pltpu.VMEM((8, 128), i32),
                   pltpu.VMEM((8, 128), i32), pltpu.VMEM((8, 128), f32),
                   pltpu.VMEM((16,), f32)],
    compiler_params=_SC_PARAMS)


def _bmean_body(xnew, batch, po, gacc, rbuf, bb):
  c = lax.axis_index("c")
  s = lax.axis_index("s")
  wid = s * NC + c
  z = jnp.zeros((16,), f32)

  def zr(i, _):
    gacc[pl.ds(i * 16, 16)] = z
    return 0

  lax.fori_loop(0, G * 80 // 16, zr, 0)
  onev = jnp.where(lax.iota(i32, 16) == 0, 1.0, 0.0).astype(f32)

  def chunkq(q, _):
    i = wid + q * 32

    @pl.when(i < 125)
    def _():
      pltpu.sync_copy(batch.at[pl.ds(i * 400, 400)], bb)
      pltpu.sync_copy(xnew.at[pl.ds(i * 400, 400)], rbuf)

      def ngrp(g, _):
        bv = bb[pl.ds(g * 16, 16)]
        for l in range(16):
          b = bv[l]
          base = b * 80
          e = g * 16 + l
          for cc in range(4):
            sl = pl.ds(base + cc * 16, 16)
            gacc[sl] = gacc[sl] + rbuf[e, pl.ds(cc * 16, 16)]
          slc = pl.ds(base + 64, 16)
          gacc[slc] = gacc[slc] + onev
        return 0

      lax.fori_loop(0, 25, ngrp, 0)
    return 0

  lax.fori_loop(0, 4, chunkq, 0)
  pltpu.sync_copy(gacc, po.at[pl.ds(wid * (G * 80), G * 80)])


_bmean = pl.kernel(
    _bmean_body,
    out_type=jax.ShapeDtypeStruct((NW * G * 80,), f32),
    mesh=_mesh,
    scratch_types=[pltpu.VMEM((G * 80,), f32), pltpu.VMEM((400, 64), f32),
                   pltpu.VMEM((400,), i32)],
    compiler_params=_SC_PARAMS)


def _expand_body(ug, batch, ubx, ugv, obuf, bb):
  c = lax.axis_index("c")
  s = lax.axis_index("s")
  wid = s * NC + c
  pltpu.sync_copy(ug, ugv)

  def chunkq(q, _):
    i = wid + q * 32

    @pl.when(i < 125)
    def _():
      pltpu.sync_copy(batch.at[pl.ds(i * 400, 400)], bb)

      def ngrp(g, _):
        bv = bb[pl.ds(g * 16, 16)]
        for l in range(16):
          b = bv[l]
          base = b * 64
          e = g * 16 + l
          for cc in range(4):
            obuf[e, pl.ds(cc * 16, 16)] = ugv[pl.ds(base + cc * 16, 16)]
        return 0

      lax.fori_loop(0, 25, ngrp, 0)
      pltpu.sync_copy(obuf, ubx.at[pl.ds(i * 400, 400)])
    return 0

  lax.fori_loop(0, 4, chunkq, 0)


_expand = pl.kernel(
    _expand_body,
    out_type=jax.ShapeDtypeStruct((N, 64), f32),
    mesh=_mesh,
    scratch_types=[pltpu.VMEM((G * 64,), f32), pltpu.VMEM((400, 64), f32),
                   pltpu.VMEM((400,), i32)],
    compiler_params=_SC_PARAMS)


BN = 2000
GRID = N // BN


def _dot(a, b):
  return jnp.dot(a, b, preferred_element_type=f32)


def _c1_body(s16, t16, w1x, wt, w2a, b2, o):
  t = t16[...]
  o1 = (_dot(s16[...], w1x[...]) + _dot(t, wt[...]))
  o1 = o1 / jnp.maximum(t[:, 4:5], 1.0)
  o[...] = _dot(o1, w2a[...]) + b2[...]


_tc_c1 = pl.pallas_call(
    _c1_body,
    grid=(GRID,),
    in_specs=[pl.BlockSpec((BN, 16), lambda i: (i, 0)),
              pl.BlockSpec((BN, 16), lambda i: (i, 0)),
              pl.BlockSpec((16, 64), lambda i: (0, 0)),
              pl.BlockSpec((16, 64), lambda i: (0, 0)),
              pl.BlockSpec((64, 64), lambda i: (0, 0)),
              pl.BlockSpec((1, 64), lambda i: (0, 0))],
    out_specs=pl.BlockSpec((BN, 64), lambda i: (i, 0)),
    out_shape=jax.ShapeDtypeStruct((N, 64), f32))


def _gatdense_body(outp, x16, w64, wx16, a2, hh1, hh2, aa, pmax):
  h = _dot(outp[...], w64[...]) + _dot(x16[...], wx16[...])
  av = _dot(h, a2[...])
  hh1[...] = h[:, :48]
  hh2[...] = jnp.concatenate(
      [h[:, 48:], jnp.ones((BN, 1), f32), jnp.zeros((BN, 15), f32)], axis=1)
  aa[...] = av
  m0 = jnp.max(av[:, 0])
  m1 = jnp.max(av[:, 1])
  pmax[...] = jnp.concatenate(
      [jnp.full((1, 1, 1), m0, f32), jnp.full((1, 1, 1), m1, f32),
       jnp.zeros((1, 1, 126), f32)], axis=2)


_tc_gatdense = pl.pallas_call(
    _gatdense_body,
    grid=(GRID,),
    in_specs=[pl.BlockSpec((BN, 64), lambda i: (i, 0)),
              pl.BlockSpec((BN, 16), lambda i: (i, 0)),
              pl.BlockSpec((64, 64), lambda i: (0, 0)),
              pl.BlockSpec((16, 64), lambda i: (0, 0)),
              pl.BlockSpec((64, 2), lambda i: (0, 0))],
    out_specs=[pl.BlockSpec((BN, 48), lambda i: (i, 0)),
               pl.BlockSpec((BN, 32), lambda i: (i, 0)),
               pl.BlockSpec((BN, 2), lambda i: (i, 0)),
               pl.BlockSpec((1, 1, 128), lambda i: (i, 0, 0))],
    out_shape=[jax.ShapeDtypeStruct((N, 48), f32),
               jax.ShapeDtypeStruct((N, 32), f32),
               jax.ShapeDtypeStruct((N, 2), f32),
               jax.ShapeDtypeStruct((GRID, 1, 128), f32)])


def _gout_body(s48, s32, bg, o1, o2):
  a = jnp.concatenate([s48[...], s32[..., :16]], axis=1)
  den = s32[..., 16:17]
  g = a / jnp.maximum(den, 1e-16) + bg[...]
  o1[...] = g[:, :32]
  o2[...] = g[:, 32:]


_tc_gout = pl.pallas_call(
    _gout_body,
    grid=(GRID,),
    in_specs=[pl.BlockSpec((BN, 48), lambda i: (i, 0)),
              pl.BlockSpec((BN, 32), lambda i: (i, 0)),
              pl.BlockSpec((1, 64), lambda i: (0, 0))],
    out_specs=[pl.BlockSpec((BN, 32), lambda i: (i, 0)),
               pl.BlockSpec((BN, 32), lambda i: (i, 0))],
    out_shape=[jax.ShapeDtypeStruct((N, 32), f32),
               jax.ShapeDtypeStruct((N, 32), f32)])


def _myc_body(sm1, sm2, t16, ubx, outp, wa, wt, w2a, b2, xn, on):
  t = t16[...]
  o1 = _dot(jnp.concatenate([sm1[...], sm2[...]], axis=1), wa[...]) \
      + _dot(t, wt[...])
  o1 = o1 / jnp.maximum(t[:, 4:5], 1.0)
  x = _dot(o1, w2a[...]) + ubx[...] + b2[...]
  xn[...] = x
  on[...] = outp[...] + x


_tc_myc = pl.pallas_call(
    _myc_body,
    grid=(GRID,),
    in_specs=[pl.BlockSpec((BN, 32), lambda i: (i, 0)),
              pl.BlockSpec((BN, 32), lambda i: (i, 0)),
              pl.BlockSpec((BN, 16), lambda i: (i, 0)),
              pl.BlockSpec((BN, 64), lambda i: (i, 0)),
              pl.BlockSpec((BN, 64), lambda i: (i, 0)),
              pl.BlockSpec((64, 64), lambda i: (0, 0)),
              pl.BlockSpec((16, 64), lambda i: (0, 0)),
              pl.BlockSpec((64, 64), lambda i: (0, 0)),
              pl.BlockSpec((1, 64), lambda i: (0, 0))],
    out_specs=[pl.BlockSpec((BN, 64), lambda i: (i, 0)),
               pl.BlockSpec((BN, 64), lambda i: (i, 0))],
    out_shape=[jax.ShapeDtypeStruct((N, 64), f32),
               jax.ShapeDtypeStruct((N, 64), f32)])


def _uupd_body(p, u, wg, bg, w2bn, gn, ugn):
  ps = jnp.sum(p[...], axis=0)
  sb = ps[:, :64] / jnp.maximum(ps[:, 64:65], 1.0)
  uu = u[...]
  unew = _dot(jnp.concatenate([uu, sb], axis=1), wg[...]) + bg[...]
  g = uu + unew
  gn[...] = g
  ugn[...] = _dot(g, w2bn[...])


_tc_uupd = pl.pallas_call(
    _uupd_body,
    in_specs=[pl.BlockSpec((NW, G, 80), lambda: (0, 0, 0)),
              pl.BlockSpec((G, 64), lambda: (0, 0)),
              pl.BlockSpec((128, 64), lambda: (0, 0)),
              pl.BlockSpec((1, 64), lambda: (0, 0)),
              pl.BlockSpec((64, 64), lambda: (0, 0))],
    out_specs=[pl.BlockSpec((G, 64), lambda: (0, 0)),
               pl.BlockSpec((G, 64), lambda: (0, 0))],
    out_shape=[jax.ShapeDtypeStruct((G, 64), f32),
               jax.ShapeDtypeStruct((G, 64), f32)])


def _head_body(g, w1, b1, w2, b2, w3, b3, o):
  p = _dot(g[...], w1[...]) + b1[...]
  p = jnp.where(p >= 0.0, p, 0.01 * p)
  p = _dot(p, w2[...]) + b2[...]
  p = jnp.where(p >= 0.0, p, 0.01 * p)
  o[...] = _dot(p, w3[...]) + b3[...]


_tc_head = pl.pallas_call(
    _head_body,
    in_specs=[pl.BlockSpec((G, 64), lambda: (0, 0)),
              pl.BlockSpec((64, 64), lambda: (0, 0)),
              pl.BlockSpec((1, 64), lambda: (0, 0)),
              pl.BlockSpec((64, 32), lambda: (0, 0)),
              pl.BlockSpec((1, 32), lambda: (0, 0)),
              pl.BlockSpec((32, 1), lambda: (0, 0)),
              pl.BlockSpec((1, 1), lambda: (0, 0))],
    out_specs=pl.BlockSpec((G, 1), lambda: (0, 0)),
    out_shape=jax.ShapeDtypeStruct((G, 1), f32))


def kernel(x, edge_index, edge_attr, batch, y, params):
  row = edge_index[0].astype(i32)
  col = edge_index[1].astype(i32)
  pad_g = (jnp.arange(PADE, dtype=i32) * 61) % N
  pad_s = jnp.full((PADE,), BIG, i32)
  row_g = jnp.concatenate([row, pad_g]).reshape(ER, 128)
  col_g = jnp.concatenate([col, pad_g]).reshape(ER, 128)
  row_s = jnp.concatenate([row, pad_s]).reshape(ER, 128)
  col_s = jnp.concatenate([col, pad_s]).reshape(ER, 128)

  ea16 = jnp.zeros((EPAD, 16), f32)
  ea16 = ea16.at[:E, :4].set(edge_attr).at[:E, 4].set(1.0)
  x16 = jnp.zeros((N, 16), f32).at[:, :2].set(x)

  t16 = _ep_lin16(ea16, row_s)
  s16 = _ep_gs16(x16, col_g, row_s)

  p1 = params["conv1"]
  w1x16 = jnp.zeros((16, 64), f32).at[:2].set(p1["W1"][:2])
  wt1 = jnp.zeros((16, 64), f32).at[:4].set(p1["W1"][2:6]).at[4].set(p1["b1"])
  out = _tc_c1(s16, t16, w1x16, wt1, p1["W2"][:64], p1["b2"][None])

  pacc = _bmean(out, batch).reshape(NW, G, 80)
  glob, ug = _tc_uupd(pacc, jnp.zeros((G, 64), f32), p1["Wg"], p1["bg"][None],
                      params["convs"][0]["W2"][64:])

  for i in range(4):
    gp = params["gats"][i]
    mp = params["convs"][i]
    wg64 = gp["W"][:64]
    wgx16 = jnp.zeros((16, 64), f32).at[:2].set(gp["W"][64:66])
    a2 = jnp.stack([gp["asrc"], gp["adst"]], axis=1)
    hh1, hh2, aa, pmax = _tc_gatdense(out, x16, wg64, wgx16, a2)
    mg = jax.nn.leaky_relu(jnp.max(pmax[:, 0, 0]) + jnp.max(pmax[:, 0, 1]),
                           0.2)
    mg16 = jnp.full((16,), mg, f32)
    ex = _gatp1(aa.reshape(-1), row_g, col_g, mg16)
    s48 = _ep_gs48s(hh1, row_g, col_s, ex)
    s32 = _ep_gs32s(hh2, row_g, col_s, ex)
    g1, g2 = _tc_gout(s48, s32, gp["b"][None])
    sm1 = _ep_gs32(g1, col_g, row_s)
    sm2 = _ep_gs32(g2, col_g, row_s)
    ubx = _expand(ug.reshape(-1), batch)
    wtm = (jnp.zeros((16, 64), f32).at[:4].set(mp["W1"][64:68])
           .at[4].set(mp["b1"]))
    xnew, out = _tc_myc(sm1, sm2, t16, ubx, out, mp["W1"][:64], wtm,
                        mp["W2"][:64], mp["b2"][None])
    pacc = _bmean(xnew, batch).reshape(NW, G, 80)
    w2bn = (params["convs"][i + 1]["W2"][64:] if i < 3
            else jnp.zeros((64, 64), f32))
    glob, ug = _tc_uupd(pacc, glob, mp["Wg"], mp["bg"][None], w2bn)

  po = params["out"]
  pred = _tc_head(glob, po["W1"], po["b1"][None], po["W2"], po["b2"][None],
                  po["W3"], po["b3"][None])
  return jnp.squeeze(pred, axis=-1)


# speedup vs baseline: 13.9430x; 1.0737x over previous
import functools

import jax
import jax.numpy as jnp
from jax import lax
from jax.experimental import pallas as pl
from jax.experimental.pallas import tpu as pltpu
from jax.experimental.pallas import tpu_sc as plsc

N = 50000
E = 800000
G = 512
C = 64

NC = 2
NS = 16
NW = NC * NS

EPAD = 819200
ER = EPAD // 128
PADE = EPAD - E
BIG = 2 ** 30

HALF = N // 2
HPAD = 25088
STRIPE = HPAD // NS

_mesh = plsc.VectorSubcoreMesh(core_axis_name="c", subcore_axis_name="s",
                               num_cores=NC, num_subcores=NS)
_SC_PARAMS = pltpu.CompilerParams(use_tc_tiling_on_sc=False,
                                  needs_layout_passes=False)

f32 = jnp.float32
i32 = jnp.int32


def _zero_rows(buf, nrows, d):
  z = jnp.zeros((16,), f32)

  def zr(i, _):
    for cc in range(d // 16):
      buf[i, pl.ds(cc * 16, 16)] = z
    return 0

  lax.fori_loop(0, nrows, zr, 0)


def _edge_pass_body(d, crows, linear_src, scale, src, *rest):
  if linear_src:
    (sidx, out, gbuf, sidxb, locb, semg, sems, acc) = rest
    gidx = exh = exb = None
  else:
    if scale:
      (gidx, sidx, exh, out, gbuf, gidxb, sidxb, locb, exb, semg, sems,
       acc) = rest
    else:
      (gidx, sidx, out, gbuf, gidxb, sidxb, locb, semg, sems, acc) = rest
      exh = exb = None

  c = lax.axis_index("c")
  s = lax.axis_index("s")
  lo = c * HALF
  it = lax.iota(i32, 16)
  zr = crows * 128

  _zero_rows(gbuf, zr, d)
  base = s * STRIPE
  for q in range(STRIPE // zr):
    pltpu.sync_copy(gbuf, acc.at[pl.ds(base + q * zr, zr)])
  rem = STRIPE % zr
  if rem:
    pltpu.sync_copy(gbuf.at[pl.ds(0, rem)],
                    acc.at[pl.ds(base + (STRIPE // zr) * zr, rem)])
  plsc.subcore_barrier()

  def chunk(k, _):
    rowb = s * 400 + k * crows
    pltpu.sync_copy(sidx.at[pl.ds(rowb, crows)], sidxb)
    if linear_src:
      pltpu.sync_copy(src.at[pl.ds(rowb * 128, zr)], gbuf)
      gd = []
    else:
      pltpu.sync_copy(gidx.at[pl.ds(rowb * 128, zr)], gidxb)
      gd = [pltpu.async_copy(src.at[gidxb], gbuf, semg)]
    if scale:
      pltpu.sync_copy(exh.at[pl.ds(rowb, crows)], exb)

      def sg(gi, _):
        j = gi // 8
        gg = (gi % 8) * 16
        ev = exb[j, pl.ds(gg, 16)]
        for l in range(16):
          val = ev[l]
          r = j * 128 + gg + l
          for cc in range(d // 16):
            sl = pl.ds(cc * 16, 16)
            gbuf[r, sl] = gbuf[r, sl] * jnp.full((16,), val)
        return 0

    def rg(g, _):
      j = g // 8
      t = (g % 8) * 16
      r = sidxb[j, pl.ds(t, 16)]
      loc = r - lo
      ok = (loc >= 0) & (loc < HALF)
      garb = HALF + ((g * 16 + it) & 63)
      locb[j, pl.ds(t, 16)] = jnp.where(ok, loc, garb)
      return 0

    lax.fori_loop(0, crows * 8, rg, 0)
    for dsc in gd:
      dsc.wait()
    if scale:
      lax.fori_loop(0, crows * 8, sg, 0)
    sd = [pltpu.async_copy(gbuf.at[pl.ds(j * 128, 128)], acc.at[locb.at[j]],
                           sems, add=True) for j in range(crows)]
    for dsc in sd:
      dsc.wait()
    return 0

  lax.fori_loop(0, 400 // crows, chunk, 0)
  plsc.subcore_barrier()

  nfull = HALF // zr
  wrem = HALF % zr

  def wb(i, _):
    @pl.when((i % 16) == s)
    def _():
      pltpu.sync_copy(acc.at[pl.ds(i * zr, zr)], gbuf)
      pltpu.sync_copy(gbuf, out.at[pl.ds(lo + i * zr, zr)])
    return 0

  lax.fori_loop(0, nfull, wb, 0)
  if wrem:
    @pl.when(s == 15)
    def _():
      pltpu.sync_copy(acc.at[pl.ds(nfull * zr, wrem)],
                      gbuf.at[pl.ds(0, wrem)])
      pltpu.sync_copy(gbuf.at[pl.ds(0, wrem)],
                      out.at[pl.ds(lo + nfull * zr, wrem)])


def _make_edge_pass(d, crows, linear_src, scale):
  scratch = [pltpu.VMEM((crows * 128, d), f32)]
  if not linear_src:
    scratch.append(pltpu.VMEM((crows * 128,), i32))
  scratch.append(pltpu.VMEM((crows, 128), i32))
  scratch.append(pltpu.VMEM((crows, 128), i32))
  if scale:
    scratch.append(pltpu.VMEM((crows, 128), f32))
  scratch.append(pltpu.SemaphoreType.DMA)
  scratch.append(pltpu.SemaphoreType.DMA)
  scratch.append(pltpu.VMEM_SHARED((HPAD, d), f32))
  return pl.kernel(
      functools.partial(_edge_pass_body, d, crows, linear_src, scale),
      out_type=jax.ShapeDtypeStruct((N, d), f32),
      mesh=_mesh, scratch_types=scratch,
      compiler_params=_SC_PARAMS)


_ep_lin16 = _make_edge_pass(16, 16, True, False)
_ep_gs16 = _make_edge_pass(16, 16, False, False)
_ep_gs32 = _make_edge_pass(32, 16, False, False)
_ep_gs48s = _make_edge_pass(48, 8, False, True)
_ep_gs32s = _make_edge_pass(32, 16, False, True)


def _gatp1_body(aa, gidx, didx, mg, exo, aav, rb, cb, eb, mgv):
  c = lax.axis_index("c")
  s = lax.axis_index("s")
  wid = s * NC + c
  pltpu.sync_copy(aa, aav)
  pltpu.sync_copy(mg, mgv)
  mgvec = mgv[...]
  rowbase = wid * 200

  def chunk(k, _):
    rowb = rowbase + k * 8
    pltpu.sync_copy(gidx.at[pl.ds(rowb, 8)], rb)
    pltpu.sync_copy(didx.at[pl.ds(rowb, 8)], cb)

    def g(gi, _):
      j = gi // 8
      t = (gi % 8) * 16
      r = rb[j, pl.ds(t, 16)]
      cc = cb[j, pl.ds(t, 16)]
      a_s = plsc.load_gather(aav, [r * 2])
      a_d = plsc.load_gather(aav, [cc * 2 + 1])
      e = a_s + a_d
      e = jnp.where(e >= 0.0, e, e * 0.2)
      eb[j, pl.ds(t, 16)] = jnp.exp(e - mgvec)
      return 0

    lax.fori_loop(0, 64, g, 0)
    pltpu.sync_copy(eb, exo.at[pl.ds(rowb, 8)])
    return 0

  lax.fori_loop(0, 25, chunk, 0)


_gatp1 = pl.kernel(
    _gatp1_body,
    out_type=jax.ShapeDtypeStruct((ER, 128), f32),
    mesh=_mesh,
    scratch_types=[pltpu.VMEM((2 * N,), f32), pltpu.VMEM((8, 128), i32),
                   pltpu.VMEM((8, 128), i32), pltpu.VMEM((8, 128), f32),
                   pltpu.VMEM((16,), f32)],
    compiler_params=_SC_PARAMS)


def _bmean_body(xnew, batch, po, gacc, rbuf, bb):
  c = lax.axis_index("c")
  s = lax.axis_index("s")
  wid = s * NC + c
  z = jnp.zeros((16,), f32)

  def zr(i, _):
    gacc[pl.ds(i * 16, 16)] = z
    return 0

  lax.fori_loop(0, G * 80 // 16, zr, 0)
  onev = jnp.where(lax.iota(i32, 16) == 0, 1.0, 0.0).astype(f32)

  def chunkq(q, _):
    i = wid + q * 32

    @pl.when(i < 125)
    def _():
      pltpu.sync_copy(batch.at[pl.ds(i * 400, 400)], bb)
      pltpu.sync_copy(xnew.at[pl.ds(i * 400, 400)], rbuf)

      def ngrp(g, _):
        bv = bb[pl.ds(g * 16, 16)]
        for l in range(16):
          b = bv[l]
          base = b * 80
          e = g * 16 + l
          for cc in range(4):
            sl = pl.ds(base + cc * 16, 16)
            gacc[sl] = gacc[sl] + rbuf[e, pl.ds(cc * 16, 16)]
          slc = pl.ds(base + 64, 16)
          gacc[slc] = gacc[slc] + onev
        return 0

      lax.fori_loop(0, 25, ngrp, 0)
    return 0

  lax.fori_loop(0, 4, chunkq, 0)
  pltpu.sync_copy(gacc, po.at[pl.ds(wid * (G * 80), G * 80)])


_bmean = pl.kernel(
    _bmean_body,
    out_type=jax.ShapeDtypeStruct((NW * G * 80,), f32),
    mesh=_mesh,
    scratch_types=[pltpu.VMEM((G * 80,), f32), pltpu.VMEM((400, 64), f32),
                   pltpu.VMEM((400,), i32)],
    compiler_params=_SC_PARAMS)


def _expand_body(ug, batch, ubx, ugv, obuf, bb):
  c = lax.axis_index("c")
  s = lax.axis_index("s")
  wid = s * NC + c
  pltpu.sync_copy(ug, ugv)

  def chunkq(q, _):
    i = wid + q * 32

    @pl.when(i < 125)
    def _():
      pltpu.sync_copy(batch.at[pl.ds(i * 400, 400)], bb)

      def ngrp(g, _):
        bv = bb[pl.ds(g * 16, 16)]
        for l in range(16):
          b = bv[l]
          base = b * 64
          e = g * 16 + l
          for cc in range(4):
            obuf[e, pl.ds(cc * 16, 16)] = ugv[pl.ds(base + cc * 16, 16)]
        return 0

      lax.fori_loop(0, 25, ngrp, 0)
      pltpu.sync_copy(obuf, ubx.at[pl.ds(i * 400, 400)])
    return 0

  lax.fori_loop(0, 4, chunkq, 0)


_expand = pl.kernel(
    _expand_body,
    out_type=jax.ShapeDtypeStruct((N, 64), f32),
    mesh=_mesh,
    scratch_types=[pltpu.VMEM((G * 64,), f32), pltpu.VMEM((400, 64), f32),
                   pltpu.VMEM((400,), i32)],
    compiler_params=_SC_PARAMS)


BN = 2000
GRID = N // BN


def _dot(a, b):
  return jnp.dot(a, b, preferred_element_type=f32)


def _c1_body(s16, t16, w1x, wt, w2a, b2, o):
  t = t16[...]
  o1 = (_dot(s16[...], w1x[...]) + _dot(t, wt[...]))
  o1 = o1 / jnp.maximum(t[:, 4:5], 1.0)
  o[...] = _dot(o1, w2a[...]) + b2[...]


_tc_c1 = pl.pallas_call(
    _c1_body,
    grid=(GRID,),
    in_specs=[pl.BlockSpec((BN, 16), lambda i: (i, 0)),
              pl.BlockSpec((BN, 16), lambda i: (i, 0)),
              pl.BlockSpec((16, 64), lambda i: (0, 0)),
              pl.BlockSpec((16, 64), lambda i: (0, 0)),
              pl.BlockSpec((64, 64), lambda i: (0, 0)),
              pl.BlockSpec((1, 64), lambda i: (0, 0))],
    out_specs=pl.BlockSpec((BN, 64), lambda i: (i, 0)),
    out_shape=jax.ShapeDtypeStruct((N, 64), f32))


def _gatdense_body(outp, x16, w64, wx16, a2, hh1, hh2, aa, pmax):
  h = _dot(outp[...], w64[...]) + _dot(x16[...], wx16[...])
  av = _dot(h, a2[...])
  hh1[...] = h[:, :48]
  hh2[...] = jnp.concatenate(
      [h[:, 48:], jnp.ones((BN, 1), f32), jnp.zeros((BN, 15), f32)], axis=1)
  aa[...] = av
  m0 = jnp.max(av[:, 0])
  m1 = jnp.max(av[:, 1])
  pmax[...] = jnp.concatenate(
      [jnp.full((1, 1, 1), m0, f32), jnp.full((1, 1, 1), m1, f32),
       jnp.zeros((1, 1, 126), f32)], axis=2)


_tc_gatdense = pl.pallas_call(
    _gatdense_body,
    grid=(GRID,),
    in_specs=[pl.BlockSpec((BN, 64), lambda i: (i, 0)),
              pl.BlockSpec((BN, 16), lambda i: (i, 0)),
              pl.BlockSpec((64, 64), lambda i: (0, 0)),
              pl.BlockSpec((16, 64), lambda i: (0, 0)),
              pl.BlockSpec((64, 2), lambda i: (0, 0))],
    out_specs=[pl.BlockSpec((BN, 48), lambda i: (i, 0)),
               pl.BlockSpec((BN, 32), lambda i: (i, 0)),
               pl.BlockSpec((BN, 2), lambda i: (i, 0)),
               pl.BlockSpec((1, 1, 128), lambda i: (i, 0, 0))],
    out_shape=[jax.ShapeDtypeStruct((N, 48), f32),
               jax.ShapeDtypeStruct((N, 32), f32),
               jax.ShapeDtypeStruct((N, 2), f32),
               jax.ShapeDtypeStruct((GRID, 1, 128), f32)])


def _gout_body(s48, s32, bg, o1, o2):
  a = jnp.concatenate([s48[...], s32[..., :16]], axis=1)
  den = s32[..., 16:17]
  g = a / jnp.maximum(den, 1e-16) + bg[...]
  o1[...] = g[:, :32]
  o2[...] = g[:, 32:]


_tc_gout = pl.pallas_call(
    _gout_body,
    grid=(GRID,),
    in_specs=[pl.BlockSpec((BN, 48), lambda i: (i, 0)),
              pl.BlockSpec((BN, 32), lambda i: (i, 0)),
              pl.BlockSpec((1, 64), lambda i: (0, 0))],
    out_specs=[pl.BlockSpec((BN, 32), lambda i: (i, 0)),
               pl.BlockSpec((BN, 32), lambda i: (i, 0))],
    out_shape=[jax.ShapeDtypeStruct((N, 32), f32),
               jax.ShapeDtypeStruct((N, 32), f32)])


def _myc_body(sm1, sm2, t16, ubx, outp, wa, wt, w2a, b2, xn, on):
  t = t16[...]
  o1 = _dot(jnp.concatenate([sm1[...], sm2[...]], axis=1), wa[...]) \
      + _dot(t, wt[...])
  o1 = o1 / jnp.maximum(t[:, 4:5], 1.0)
  x = _dot(o1, w2a[...]) + ubx[...] + b2[...]
  xn[...] = x
  on[...] = outp[...] + x


_tc_myc = pl.pallas_call(
    _myc_body,
    grid=(GRID,),
    in_specs=[pl.BlockSpec((BN, 32), lambda i: (i, 0)),
              pl.BlockSpec((BN, 32), lambda i: (i, 0)),
              pl.BlockSpec((BN, 16), lambda i: (i, 0)),
              pl.BlockSpec((BN, 64), lambda i: (i, 0)),
              pl.BlockSpec((BN, 64), lambda i: (i, 0)),
              pl.BlockSpec((64, 64), lambda i: (0, 0)),
              pl.BlockSpec((16, 64), lambda i: (0, 0)),
              pl.BlockSpec((64, 64), lambda i: (0, 0)),
              pl.BlockSpec((1, 64), lambda i: (0, 0))],
    out_specs=[pl.BlockSpec((BN, 64), lambda i: (i, 0)),
               pl.BlockSpec((BN, 64), lambda i: (i, 0))],
    out_shape=[jax.ShapeDtypeStruct((N, 64), f32),
               jax.ShapeDtypeStruct((N, 64), f32)])


def _uupd_body(p, u, wg, bg, w2bn, gn, ugn):
  ps = jnp.sum(p[...], axis=0)
  sb = ps[:, :64] / jnp.maximum(ps[:, 64:65], 1.0)
  uu = u[...]
  unew = _dot(jnp.concatenate([uu, sb], axis=1), wg[...]) + bg[...]
  g = uu + unew
  gn[...] = g
  ugn[...] = _dot(g, w2bn[...])


_tc_uupd = pl.pallas_call(
    _uupd_body,
    in_specs=[pl.BlockSpec((NW, G, 80), lambda: (0, 0, 0)),
              pl.BlockSpec((G, 64), lambda: (0, 0)),
              pl.BlockSpec((128, 64), lambda: (0, 0)),
              pl.BlockSpec((1, 64), lambda: (0, 0)),
              pl.BlockSpec((64, 64), lambda: (0, 0))],
    out_specs=[pl.BlockSpec((G, 64), lambda: (0, 0)),
               pl.BlockSpec((G, 64), lambda: (0, 0))],
    out_shape=[jax.ShapeDtypeStruct((G, 64), f32),
               jax.ShapeDtypeStruct((G, 64), f32)])


def _head_body(g, w1, b1, w2, b2, w3, b3, o):
  p = _dot(g[...], w1[...]) + b1[...]
  p = jnp.where(p >= 0.0, p, 0.01 * p)
  p = _dot(p, w2[...]) + b2[...]
  p = jnp.where(p >= 0.0, p, 0.01 * p)
  o[...] = _dot(p, w3[...]) + b3[...]


_tc_head = pl.pallas_call(
    _head_body,
    in_specs=[pl.BlockSpec((G, 64), lambda: (0, 0)),
              pl.BlockSpec((64, 64), lambda: (0, 0)),
              pl.BlockSpec((1, 64), lambda: (0, 0)),
              pl.BlockSpec((64, 32), lambda: (0, 0)),
              pl.BlockSpec((1, 32), lambda: (0, 0)),
              pl.BlockSpec((32, 1), lambda: (0, 0)),
              pl.BlockSpec((1, 1), lambda: (0, 0))],
    out_specs=pl.BlockSpec((G, 1), lambda: (0, 0)),
    out_shape=jax.ShapeDtypeStruct((G, 1), f32))


def kernel(x, edge_index, edge_attr, batch, y, params):
  row = edge_index[0].astype(i32)
  col = edge_index[1].astype(i32)
  pad_g = (jnp.arange(PADE, dtype=i32) * 61) % N
  pad_s = jnp.full((PADE,), BIG, i32)
  row_g = jnp.concatenate([row, pad_g])
  col_g = jnp.concatenate([col, pad_g])
  row_g2 = row_g.reshape(ER, 128)
  col_g2 = col_g.reshape(ER, 128)
  row_s = jnp.concatenate([row, pad_s]).reshape(ER, 128)
  col_s = jnp.concatenate([col, pad_s]).reshape(ER, 128)

  ea16 = jnp.zeros((EPAD, 16), f32)
  ea16 = ea16.at[:E, :4].set(edge_attr).at[:E, 4].set(1.0)
  x16 = jnp.zeros((N, 16), f32).at[:, :2].set(x)

  t16 = _ep_lin16(ea16, row_s)
  s16 = _ep_gs16(x16, col_g, row_s)

  p1 = params["conv1"]
  w1x16 = jnp.zeros((16, 64), f32).at[:2].set(p1["W1"][:2])
  wt1 = jnp.zeros((16, 64), f32).at[:4].set(p1["W1"][2:6]).at[4].set(p1["b1"])
  out = _tc_c1(s16, t16, w1x16, wt1, p1["W2"][:64], p1["b2"][None])

  pacc = _bmean(out, batch).reshape(NW, G, 80)
  glob, ug = _tc_uupd(pacc, jnp.zeros((G, 64), f32), p1["Wg"], p1["bg"][None],
                      params["convs"][0]["W2"][64:])

  for i in range(4):
    gp = params["gats"][i]
    mp = params["convs"][i]
    wg64 = gp["W"][:64]
    wgx16 = jnp.zeros((16, 64), f32).at[:2].set(gp["W"][64:66])
    a2 = jnp.stack([gp["asrc"], gp["adst"]], axis=1)
    hh1, hh2, aa, pmax = _tc_gatdense(out, x16, wg64, wgx16, a2)
    mg = jax.nn.leaky_relu(jnp.max(pmax[:, 0, 0]) + jnp.max(pmax[:, 0, 1]),
                           0.2)
    mg16 = jnp.full((16,), mg, f32)
    ex = _gatp1(aa.reshape(-1), row_g2, col_g2, mg16)
    s48 = _ep_gs48s(hh1, row_g, col_s, ex)
    s32 = _ep_gs32s(hh2, row_g, col_s, ex)
    g1, g2 = _tc_gout(s48, s32, gp["b"][None])
    sm1 = _ep_gs32(g1, col_g, row_s)
    sm2 = _ep_gs32(g2, col_g, row_s)
    ubx = _expand(ug.reshape(-1), batch)
    wtm = (jnp.zeros((16, 64), f32).at[:4].set(mp["W1"][64:68])
           .at[4].set(mp["b1"]))
    xnew, out = _tc_myc(sm1, sm2, t16, ubx, out, mp["W1"][:64], wtm,
                        mp["W2"][:64], mp["b2"][None])
    pacc = _bmean(xnew, batch).reshape(NW, G, 80)
    w2bn = (params["convs"][i + 1]["W2"][64:] if i < 3
            else jnp.zeros((64, 64), f32))
    glob, ug = _tc_uupd(pacc, glob, mp["Wg"], mp["bg"][None], w2bn)

  po = params["out"]
  pred = _tc_head(glob, po["W1"], po["b1"][None], po["W2"], po["b2"][None],
                  po["W3"], po["b3"][None])
  return jnp.squeeze(pred, axis=-1)


# speedup vs baseline: 14.8848x; 1.0675x over previous
import functools

import jax
import jax.numpy as jnp
from jax import lax
from jax.experimental import pallas as pl
from jax.experimental.pallas import tpu as pltpu
from jax.experimental.pallas import tpu_sc as plsc

N = 50000
E = 800000
G = 512
C = 64

NC = 2
NS = 16
NW = NC * NS

EPAD = 819200
ER = EPAD // 128
PADE = EPAD - E
BIG = 2 ** 30

HALF = N // 2
HPAD = 25088
STRIPE = HPAD // NS

_mesh = plsc.VectorSubcoreMesh(core_axis_name="c", subcore_axis_name="s",
                               num_cores=NC, num_subcores=NS)
_SC_PARAMS = pltpu.CompilerParams(use_tc_tiling_on_sc=False,
                                  needs_layout_passes=False)

f32 = jnp.float32
i32 = jnp.int32


def _zero_rows(buf, nrows, d):
  z = jnp.zeros((16,), f32)

  def zr(i, _):
    for cc in range(d // 16):
      buf[i, pl.ds(cc * 16, 16)] = z
    return 0

  lax.fori_loop(0, nrows, zr, 0)


def _edge_pass_body(d, crows, linear_src, scale, src, *rest):
  it = rest
  n_extra = (0 if linear_src else 2) + (2 if scale else 0)
  nin = (1 if linear_src else 2) + (1 if scale else 0)
  ins, rest = it[:nin], it[nin:]
  if linear_src:
    sidx = ins[0]
    gidx = ins[1] if scale else None
    exh = ins[1] if scale else None
  else:
    gidx, sidx = ins[0], ins[1]
    exh = ins[2] if scale else None
  out = rest[0]
  sc = list(rest[1:])
  gbuf = [sc.pop(0), sc.pop(0)]
  if linear_src:
    gidxb = None
  else:
    gidxb = [sc.pop(0), sc.pop(0)]
  sidxb = [sc.pop(0), sc.pop(0)]
  locb = [sc.pop(0), sc.pop(0)]
  if scale:
    exb = [sc.pop(0), sc.pop(0)]
  else:
    exb = None
  semg = [sc.pop(0), sc.pop(0)]
  sems = [sc.pop(0), sc.pop(0)]
  acc = sc.pop(0)

  c = lax.axis_index("c")
  s = lax.axis_index("s")
  lo = c * HALF
  iv = lax.iota(i32, 16)
  zr = crows * 128
  nch = 400 // crows

  _zero_rows(gbuf[0], zr, d)
  base = s * STRIPE
  for q in range(STRIPE // zr):
    pltpu.sync_copy(gbuf[0], acc.at[pl.ds(base + q * zr, zr)])
  rem = STRIPE % zr
  if rem:
    pltpu.sync_copy(gbuf[0].at[pl.ds(0, rem)],
                    acc.at[pl.ds(base + (STRIPE // zr) * zr, rem)])
  plsc.subcore_barrier()

  def load_fire(k, nb):
    rowb = s * 400 + k * crows
    pltpu.sync_copy(sidx.at[pl.ds(rowb, crows)], sidxb[nb])
    if scale:
      pltpu.sync_copy(exh.at[pl.ds(rowb, crows)], exb[nb])
    if linear_src:
      pltpu.async_copy(src.at[pl.ds(rowb * 128, zr)], gbuf[nb], semg[nb])
    else:
      pltpu.sync_copy(gidx.at[pl.ds(rowb * 128, zr)], gidxb[nb])
      pltpu.async_copy(src.at[gidxb[nb]], gbuf[nb], semg[nb])

  def drain_gather(nb):
    if linear_src:
      pltpu.make_async_copy(src.at[pl.ds(0, zr)], gbuf[nb], semg[nb]).wait()
    else:
      pltpu.make_async_copy(src.at[gidxb[nb]], gbuf[nb], semg[nb]).wait()

  def drain_scatter(nb):
    pltpu.make_async_copy(gbuf[nb], acc.at[pl.ds(0, zr)], sems[nb]).wait()

  def remap(b):
    def rg(g, _):
      j = g // 8
      t = (g % 8) * 16
      r = sidxb[b][j, pl.ds(t, 16)]
      loc = r - lo
      ok = (loc >= 0) & (loc < HALF)
      garb = HALF + ((g * 16 + iv) & 63)
      locb[b][j, pl.ds(t, 16)] = jnp.where(ok, loc, garb)
      return 0

    lax.fori_loop(0, crows * 8, rg, 0)

  def do_scale(b):
    def sg(gi, _):
      j = gi // 8
      gg = (gi % 8) * 16
      ev = exb[b][j, pl.ds(gg, 16)]
      for l in range(16):
        val = ev[l]
        r = j * 128 + gg + l
        for cc in range(d // 16):
          sl = pl.ds(cc * 16, 16)
          gbuf[b][r, sl] = gbuf[b][r, sl] * jnp.full((16,), val)
      return 0

    lax.fori_loop(0, crows * 8, sg, 0)

  def step(k, b):
    nb = 1 - b

    @pl.when(k + 1 < nch)
    def _():
      @pl.when(k >= 1)
      def _():
        drain_scatter(nb)
      load_fire(k + 1, nb)

    remap(b)
    drain_gather(b)
    if scale:
      do_scale(b)
    for j in range(crows):
      pltpu.async_copy(gbuf[b].at[pl.ds(j * 128, 128)],
                       acc.at[locb[b].at[j]], sems[b], add=True)

  load_fire(0, 0)

  def loop(k2, _):
    step(2 * k2, 0)
    step(2 * k2 + 1, 1)
    return 0

  lax.fori_loop(0, nch // 2, loop, 0)
  drain_scatter(0)
  drain_scatter(1)
  plsc.subcore_barrier()

  nfull = HALF // zr
  wrem = HALF % zr

  def wb(i, _):
    @pl.when((i % 16) == s)
    def _():
      pltpu.sync_copy(acc.at[pl.ds(i * zr, zr)], gbuf[0])
      pltpu.sync_copy(gbuf[0], out.at[pl.ds(lo + i * zr, zr)])
    return 0

  lax.fori_loop(0, nfull, wb, 0)
  if wrem:
    @pl.when(s == 15)
    def _():
      pltpu.sync_copy(acc.at[pl.ds(nfull * zr, wrem)],
                      gbuf[0].at[pl.ds(0, wrem)])
      pltpu.sync_copy(gbuf[0].at[pl.ds(0, wrem)],
                      out.at[pl.ds(lo + nfull * zr, wrem)])


def _make_edge_pass(d, crows, linear_src, scale):
  scratch = [pltpu.VMEM((crows * 128, d), f32),
             pltpu.VMEM((crows * 128, d), f32)]
  if not linear_src:
    scratch += [pltpu.VMEM((crows * 128,), i32)] * 2
  scratch += [pltpu.VMEM((crows, 128), i32)] * 2
  scratch += [pltpu.VMEM((crows, 128), i32)] * 2
  if scale:
    scratch += [pltpu.VMEM((crows, 128), f32)] * 2
  scratch += [pltpu.SemaphoreType.DMA] * 4
  scratch.append(pltpu.VMEM_SHARED((HPAD, d), f32))
  return pl.kernel(
      functools.partial(_edge_pass_body, d, crows, linear_src, scale),
      out_type=jax.ShapeDtypeStruct((N, d), f32),
      mesh=_mesh, scratch_types=scratch,
      compiler_params=_SC_PARAMS)


_ep_lin16 = _make_edge_pass(16, 8, True, False)
_ep_gs16 = _make_edge_pass(16, 8, False, False)
_ep_gs32 = _make_edge_pass(32, 8, False, False)
_ep_gs48s = _make_edge_pass(48, 4, False, True)
_ep_gs32s = _make_edge_pass(32, 8, False, True)


def _gatp1_body(aa, gidx, didx, mg, exo, aav, rb, cb, eb, mgv):
  c = lax.axis_index("c")
  s = lax.axis_index("s")
  wid = s * NC + c
  pltpu.sync_copy(aa, aav)
  pltpu.sync_copy(mg, mgv)
  mgvec = mgv[...]
  rowbase = wid * 200

  def chunk(k, _):
    rowb = rowbase + k * 8
    pltpu.sync_copy(gidx.at[pl.ds(rowb, 8)], rb)
    pltpu.sync_copy(didx.at[pl.ds(rowb, 8)], cb)

    def g(gi, _):
      j = gi // 8
      t = (gi % 8) * 16
      r = rb[j, pl.ds(t, 16)]
      cc = cb[j, pl.ds(t, 16)]
      a_s = plsc.load_gather(aav, [r * 2])
      a_d = plsc.load_gather(aav, [cc * 2 + 1])
      e = a_s + a_d
      e = jnp.where(e >= 0.0, e, e * 0.2)
      eb[j, pl.ds(t, 16)] = e - mgvec
      return 0

    lax.fori_loop(0, 64, g, 0)
    pltpu.sync_copy(eb, exo.at[pl.ds(rowb, 8)])
    return 0

  lax.fori_loop(0, 25, chunk, 0)


_gatp1 = pl.kernel(
    _gatp1_body,
    out_type=jax.ShapeDtypeStruct((ER, 128), f32),
    mesh=_mesh,
    scratch_types=[pltpu.VMEM((2 * N,), f32), pltpu.VMEM((8, 128), i32),
                   pltpu.VMEM((8, 128), i32), pltpu.VMEM((8, 128), f32),
                   pltpu.VMEM((16,), f32)],
    compiler_params=_SC_PARAMS)


def _bmean_body(xnew, batch, po, gacc, rbuf, bb):
  c = lax.axis_index("c")
  s = lax.axis_index("s")
  wid = s * NC + c
  z = jnp.zeros((16,), f32)

  def zr(i, _):
    gacc[pl.ds(i * 16, 16)] = z
    return 0

  lax.fori_loop(0, G * 80 // 16, zr, 0)
  onev = jnp.where(lax.iota(i32, 16) == 0, 1.0, 0.0).astype(f32)

  def chunkq(q, _):
    i = wid + q * 32

    @pl.when(i < 125)
    def _():
      pltpu.sync_copy(batch.at[pl.ds(i * 400, 400)], bb)
      pltpu.sync_copy(xnew.at[pl.ds(i * 400, 400)], rbuf)

      def ngrp(g, _):
        bv = bb[pl.ds(g * 16, 16)]
        for l in range(16):
          b = bv[l]
          base = b * 80
          e = g * 16 + l
          for cc in range(4):
            sl = pl.ds(base + cc * 16, 16)
            gacc[sl] = gacc[sl] + rbuf[e, pl.ds(cc * 16, 16)]
          slc = pl.ds(base + 64, 16)
          gacc[slc] = gacc[slc] + onev
        return 0

      lax.fori_loop(0, 25, ngrp, 0)
    return 0

  lax.fori_loop(0, 4, chunkq, 0)
  pltpu.sync_copy(gacc, po.at[pl.ds(wid * (G * 80), G * 80)])


_bmean = pl.kernel(
    _bmean_body,
    out_type=jax.ShapeDtypeStruct((NW * G * 80,), f32),
    mesh=_mesh,
    scratch_types=[pltpu.VMEM((G * 80,), f32), pltpu.VMEM((400, 64), f32),
                   pltpu.VMEM((400,), i32)],
    compiler_params=_SC_PARAMS)


def _expand_body(ug, batch, ubx, ugv, obuf, bb):
  c = lax.axis_index("c")
  s = lax.axis_index("s")
  wid = s * NC + c
  pltpu.sync_copy(ug, ugv)

  def chunkq(q, _):
    i = wid + q * 32

    @pl.when(i < 125)
    def _():
      pltpu.sync_copy(batch.at[pl.ds(i * 400, 400)], bb)

      def ngrp(g, _):
        bv = bb[pl.ds(g * 16, 16)]
        for l in range(16):
          b = bv[l]
          base = b * 64
          e = g * 16 + l
          for cc in range(4):
            obuf[e, pl.ds(cc * 16, 16)] = ugv[pl.ds(base + cc * 16, 16)]
        return 0

      lax.fori_loop(0, 25, ngrp, 0)
      pltpu.sync_copy(obuf, ubx.at[pl.ds(i * 400, 400)])
    return 0

  lax.fori_loop(0, 4, chunkq, 0)


_expand = pl.kernel(
    _expand_body,
    out_type=jax.ShapeDtypeStruct((N, 64), f32),
    mesh=_mesh,
    scratch_types=[pltpu.VMEM((G * 64,), f32), pltpu.VMEM((400, 64), f32),
                   pltpu.VMEM((400,), i32)],
    compiler_params=_SC_PARAMS)


BN = 2000
GRID = N // BN


def _dot(a, b):
  return jnp.dot(a, b, preferred_element_type=f32)


def _c1_body(s16, t16, w1x, wt, w2a, b2, o):
  t = t16[...]
  o1 = (_dot(s16[...], w1x[...]) + _dot(t, wt[...]))
  o1 = o1 / jnp.maximum(t[:, 4:5], 1.0)
  o[...] = _dot(o1, w2a[...]) + b2[...]


_tc_c1 = pl.pallas_call(
    _c1_body,
    grid=(GRID,),
    in_specs=[pl.BlockSpec((BN, 16), lambda i: (i, 0)),
              pl.BlockSpec((BN, 16), lambda i: (i, 0)),
              pl.BlockSpec((16, 64), lambda i: (0, 0)),
              pl.BlockSpec((16, 64), lambda i: (0, 0)),
              pl.BlockSpec((64, 64), lambda i: (0, 0)),
              pl.BlockSpec((1, 64), lambda i: (0, 0))],
    out_specs=pl.BlockSpec((BN, 64), lambda i: (i, 0)),
    out_shape=jax.ShapeDtypeStruct((N, 64), f32))


def _gatdense_body(outp, x16, w64, wx16, a2, hh1, hh2, aa, pmax):
  h = _dot(outp[...], w64[...]) + _dot(x16[...], wx16[...])
  av = _dot(h, a2[...])
  hh1[...] = h[:, :48]
  hh2[...] = jnp.concatenate(
      [h[:, 48:], jnp.ones((BN, 1), f32), jnp.zeros((BN, 15), f32)], axis=1)
  aa[...] = av
  m0 = jnp.max(av[:, 0])
  m1 = jnp.max(av[:, 1])
  pmax[...] = jnp.concatenate(
      [jnp.full((1, 1, 1), m0, f32), jnp.full((1, 1, 1), m1, f32),
       jnp.zeros((1, 1, 126), f32)], axis=2)


_tc_gatdense = pl.pallas_call(
    _gatdense_body,
    grid=(GRID,),
    in_specs=[pl.BlockSpec((BN, 64), lambda i: (i, 0)),
              pl.BlockSpec((BN, 16), lambda i: (i, 0)),
              pl.BlockSpec((64, 64), lambda i: (0, 0)),
              pl.BlockSpec((16, 64), lambda i: (0, 0)),
              pl.BlockSpec((64, 2), lambda i: (0, 0))],
    out_specs=[pl.BlockSpec((BN, 48), lambda i: (i, 0)),
               pl.BlockSpec((BN, 32), lambda i: (i, 0)),
               pl.BlockSpec((BN, 2), lambda i: (i, 0)),
               pl.BlockSpec((1, 1, 128), lambda i: (i, 0, 0))],
    out_shape=[jax.ShapeDtypeStruct((N, 48), f32),
               jax.ShapeDtypeStruct((N, 32), f32),
               jax.ShapeDtypeStruct((N, 2), f32),
               jax.ShapeDtypeStruct((GRID, 1, 128), f32)])


def _gout_body(s48, s32, bg, o1, o2):
  a = jnp.concatenate([s48[...], s32[..., :16]], axis=1)
  den = s32[..., 16:17]
  g = a / jnp.maximum(den, 1e-16) + bg[...]
  o1[...] = g[:, :32]
  o2[...] = g[:, 32:]


_tc_gout = pl.pallas_call(
    _gout_body,
    grid=(GRID,),
    in_specs=[pl.BlockSpec((BN, 48), lambda i: (i, 0)),
              pl.BlockSpec((BN, 32), lambda i: (i, 0)),
              pl.BlockSpec((1, 64), lambda i: (0, 0))],
    out_specs=[pl.BlockSpec((BN, 32), lambda i: (i, 0)),
               pl.BlockSpec((BN, 32), lambda i: (i, 0))],
    out_shape=[jax.ShapeDtypeStruct((N, 32), f32),
               jax.ShapeDtypeStruct((N, 32), f32)])


def _myc_body(sm1, sm2, t16, ubx, outp, wa, wt, w2a, b2, xn, on):
  t = t16[...]
  o1 = _dot(jnp.concatenate([sm1[...], sm2[...]], axis=1), wa[...]) \
      + _dot(t, wt[...])
  o1 = o1 / jnp.maximum(t[:, 4:5], 1.0)
  x = _dot(o1, w2a[...]) + ubx[...] + b2[...]
  xn[...] = x
  on[...] = outp[...] + x


_tc_myc = pl.pallas_call(
    _myc_body,
    grid=(GRID,),
    in_specs=[pl.BlockSpec((BN, 32), lambda i: (i, 0)),
              pl.BlockSpec((BN, 32), lambda i: (i, 0)),
              pl.BlockSpec((BN, 16), lambda i: (i, 0)),
              pl.BlockSpec((BN, 64), lambda i: (i, 0)),
              pl.BlockSpec((BN, 64), lambda i: (i, 0)),
              pl.BlockSpec((64, 64), lambda i: (0, 0)),
              pl.BlockSpec((16, 64), lambda i: (0, 0)),
              pl.BlockSpec((64, 64), lambda i: (0, 0)),
              pl.BlockSpec((1, 64), lambda i: (0, 0))],
    out_specs=[pl.BlockSpec((BN, 64), lambda i: (i, 0)),
               pl.BlockSpec((BN, 64), lambda i: (i, 0))],
    out_shape=[jax.ShapeDtypeStruct((N, 64), f32),
               jax.ShapeDtypeStruct((N, 64), f32)])


def _uupd_body(p, u, wg, bg, w2bn, gn, ugn):
  ps = jnp.sum(p[...], axis=0)
  sb = ps[:, :64] / jnp.maximum(ps[:, 64:65], 1.0)
  uu = u[...]
  unew = _dot(jnp.concatenate([uu, sb], axis=1), wg[...]) + bg[...]
  g = uu + unew
  gn[...] = g
  ugn[...] = _dot(g, w2bn[...])


_tc_uupd = pl.pallas_call(
    _uupd_body,
    in_specs=[pl.BlockSpec((NW, G, 80), lambda: (0, 0, 0)),
              pl.BlockSpec((G, 64), lambda: (0, 0)),
              pl.BlockSpec((128, 64), lambda: (0, 0)),
              pl.BlockSpec((1, 64), lambda: (0, 0)),
              pl.BlockSpec((64, 64), lambda: (0, 0))],
    out_specs=[pl.BlockSpec((G, 64), lambda: (0, 0)),
               pl.BlockSpec((G, 64), lambda: (0, 0))],
    out_shape=[jax.ShapeDtypeStruct((G, 64), f32),
               jax.ShapeDtypeStruct((G, 64), f32)])


def _head_body(g, w1, b1, w2, b2, w3, b3, o):
  p = _dot(g[...], w1[...]) + b1[...]
  p = jnp.where(p >= 0.0, p, 0.01 * p)
  p = _dot(p, w2[...]) + b2[...]
  p = jnp.where(p >= 0.0, p, 0.01 * p)
  o[...] = _dot(p, w3[...]) + b3[...]


_tc_head = pl.pallas_call(
    _head_body,
    in_specs=[pl.BlockSpec((G, 64), lambda: (0, 0)),
              pl.BlockSpec((64, 64), lambda: (0, 0)),
              pl.BlockSpec((1, 64), lambda: (0, 0)),
              pl.BlockSpec((64, 32), lambda: (0, 0)),
              pl.BlockSpec((1, 32), lambda: (0, 0)),
              pl.BlockSpec((32, 1), lambda: (0, 0)),
              pl.BlockSpec((1, 1), lambda: (0, 0))],
    out_specs=pl.BlockSpec((G, 1), lambda: (0, 0)),
    out_shape=jax.ShapeDtypeStruct((G, 1), f32))


def _exp_body(e, o):
  o[...] = jnp.exp(e[...])


_tc_exp = pl.pallas_call(
    _exp_body,
    grid=(ER // 256,),
    in_specs=[pl.BlockSpec((256, 128), lambda i: (i, 0))],
    out_specs=pl.BlockSpec((256, 128), lambda i: (i, 0)),
    out_shape=jax.ShapeDtypeStruct((ER, 128), f32))


def kernel(x, edge_index, edge_attr, batch, y, params):
  row = edge_index[0].astype(i32)
  col = edge_index[1].astype(i32)
  pad_g = (jnp.arange(PADE, dtype=i32) * 61) % N
  pad_s = jnp.full((PADE,), BIG, i32)
  row_g = jnp.concatenate([row, pad_g])
  col_g = jnp.concatenate([col, pad_g])
  row_g2 = row_g.reshape(ER, 128)
  col_g2 = col_g.reshape(ER, 128)
  row_s = jnp.concatenate([row, pad_s]).reshape(ER, 128)
  col_s = jnp.concatenate([col, pad_s]).reshape(ER, 128)

  ea16 = jnp.zeros((EPAD, 16), f32)
  ea16 = ea16.at[:E, :4].set(edge_attr).at[:E, 4].set(1.0)
  x16 = jnp.zeros((N, 16), f32).at[:, :2].set(x)

  t16 = _ep_lin16(ea16, row_s)
  s16 = _ep_gs16(x16, col_g, row_s)

  p1 = params["conv1"]
  w1x16 = jnp.zeros((16, 64), f32).at[:2].set(p1["W1"][:2])
  wt1 = jnp.zeros((16, 64), f32).at[:4].set(p1["W1"][2:6]).at[4].set(p1["b1"])
  out = _tc_c1(s16, t16, w1x16, wt1, p1["W2"][:64], p1["b2"][None])

  pacc = _bmean(out, batch).reshape(NW, G, 80)
  glob, ug = _tc_uupd(pacc, jnp.zeros((G, 64), f32), p1["Wg"], p1["bg"][None],
                      params["convs"][0]["W2"][64:])

  for i in range(4):
    gp = params["gats"][i]
    mp = params["convs"][i]
    wg64 = gp["W"][:64]
    wgx16 = jnp.zeros((16, 64), f32).at[:2].set(gp["W"][64:66])
    a2 = jnp.stack([gp["asrc"], gp["adst"]], axis=1)
    hh1, hh2, aa, pmax = _tc_gatdense(out, x16, wg64, wgx16, a2)
    mg = jax.nn.leaky_relu(jnp.max(pmax[:, 0, 0]) + jnp.max(pmax[:, 0, 1]),
                           0.2)
    mg16 = jnp.full((16,), mg, f32)
    ex = _tc_exp(_gatp1(aa.reshape(-1), row_g2, col_g2, mg16))
    s48 = _ep_gs48s(hh1, row_g, col_s, ex)
    s32 = _ep_gs32s(hh2, row_g, col_s, ex)
    g1, g2 = _tc_gout(s48, s32, gp["b"][None])
    sm1 = _ep_gs32(g1, col_g, row_s)
    sm2 = _ep_gs32(g2, col_g, row_s)
    ubx = _expand(ug.reshape(-1), batch)
    wtm = (jnp.zeros((16, 64), f32).at[:4].set(mp["W1"][64:68])
           .at[4].set(mp["b1"]))
    xnew, out = _tc_myc(sm1, sm2, t16, ubx, out, mp["W1"][:64], wtm,
                        mp["W2"][:64], mp["b2"][None])
    pacc = _bmean(xnew, batch).reshape(NW, G, 80)
    w2bn = (params["convs"][i + 1]["W2"][64:] if i < 3
            else jnp.zeros((64, 64), f32))
    glob, ug = _tc_uupd(pacc, glob, mp["Wg"], mp["bg"][None], w2bn)

  po = params["out"]
  pred = _tc_head(glob, po["W1"], po["b1"][None], po["W2"], po["b2"][None],
                  po["W3"], po["b3"][None])
  return jnp.squeeze(pred, axis=-1)
